# baseline (speedup 1.0000x reference)
import functools

import jax
import jax.numpy as jnp
from jax import lax
from jax.experimental import pallas as pl
from jax.experimental.pallas import tpu as pltpu
from jax.experimental.pallas import tpu_sc as plsc

V = 50
C = 2000
H = 64
B = 1024
NW = 32
B_PER_W = B // NW
NGRP = 4
GRP = B_PER_W // NGRP


def _sc_gather_body(table_hbm, idx_hbm, out_hbm, idx_v, rows_v, gsem, wsem):
    wid = lax.axis_index("s") * 2 + lax.axis_index("c")

    def stage(g, buf):
        base = wid * B_PER_W + g * GRP
        pltpu.sync_copy(idx_hbm.at[pl.ds(base, GRP)], idx_v.at[buf])
        return [
            pltpu.async_copy(
                table_hbm.at[idx_v.at[buf, b]], rows_v.at[buf, b], gsem
            )
            for b in range(GRP)
        ]

    pending = stage(0, 0)
    wb = None
    for g in range(NGRP):
        if wb is not None:
            wb.wait()
        nxt = None
        if g + 1 < NGRP:
            nxt = stage(g + 1, (g + 1) % 2)
        for cp in pending:
            cp.wait()
        base = wid * B_PER_W + g * GRP
        wb = pltpu.async_copy(rows_v.at[g % 2], out_hbm.at[pl.ds(base, GRP)], wsem)
        pending = nxt
    wb.wait()


@functools.cache
def _sc_gather():
    return pl.kernel(
        _sc_gather_body,
        out_type=jax.ShapeDtypeStruct((B, V, 2 * H), jnp.float32),
        mesh=plsc.VectorSubcoreMesh(core_axis_name="c", subcore_axis_name="s"),
        scratch_types=[
            pltpu.VMEM((2, GRP, V), jnp.int32),
            pltpu.VMEM((2, GRP, V, 2 * H), jnp.float32),
            pltpu.SemaphoreType.DMA,
            pltpu.SemaphoreType.DMA,
        ],
    )


BB = 256


def _mm_body(x_ref, mask_ref, e2_ref, bias_ref, out_ref):
    b = bias_ref[...]
    par = (x_ref[...] & 1).astype(jnp.float32)
    dn = (((1,), (0,)), ((), ()))
    for k in range(BB):
        pk = par[k:k + 1, :]
        mk = mask_ref[k]
        m1 = mk * pk
        m0 = mk - m1
        acc = lax.dot_general(
            m0.astype(jnp.bfloat16),
            e2_ref[k, :, :H].astype(jnp.bfloat16),
            dimension_numbers=dn,
            preferred_element_type=jnp.float32,
        ) + lax.dot_general(
            m1.astype(jnp.bfloat16),
            e2_ref[k, :, H:].astype(jnp.bfloat16),
            dimension_numbers=dn,
            preferred_element_type=jnp.float32,
        )
        out_ref[k] = acc + b


def _mm(x, mask, e2, bias, interpret=False):
    return pl.pallas_call(
        _mm_body,
        grid=(B // BB,),
        in_specs=[
            pl.BlockSpec((BB, V), lambda i: (i, 0)),
            pl.BlockSpec((BB, V, V), lambda i: (i, 0, 0)),
            pl.BlockSpec((BB, V, 2 * H), lambda i: (i, 0, 0)),
            pl.BlockSpec((V, H), lambda i: (0, 0)),
        ],
        out_specs=pl.BlockSpec((BB, V, H), lambda i: (i, 0, 0)),
        out_shape=jax.ShapeDtypeStruct((B, V, H), jnp.float32),
        compiler_params=pltpu.CompilerParams(
            dimension_semantics=("parallel",),
        ),
        interpret=interpret,
    )(x, mask, e2, bias)


def kernel(x, mask, emb_table, bias):
    xi = x.astype(jnp.int32)
    pos = jnp.arange(V, dtype=jnp.int32) * C
    idx_phys = (xi + pos[None, :]) >> 1
    table2 = emb_table.reshape(C * V // 2, 2 * H)
    e2 = _sc_gather()(table2, idx_phys)
    out = _mm(xi, mask, e2, bias)
    return lax.optimization_barrier(out)


# speedup vs baseline: 1.0066x; 1.0066x over previous
import functools

import jax
import jax.numpy as jnp
from jax import lax
from jax.experimental import pallas as pl
from jax.experimental.pallas import tpu as pltpu
from jax.experimental.pallas import tpu_sc as plsc

V = 50
C = 2000
H = 64
B = 1024
NW = 32
B_PER_W = B // NW
NGRP = 4
GRP = B_PER_W // NGRP


def _sc_gather_body(table_hbm, idx_hbm, out_hbm, idx_v, rows_v, gsem, wsem):
    wid = lax.axis_index("s") * 2 + lax.axis_index("c")

    def stage(g, buf):
        base = wid * B_PER_W + g * GRP
        pltpu.sync_copy(idx_hbm.at[pl.ds(base, GRP)], idx_v.at[buf])
        return [
            pltpu.async_copy(
                table_hbm.at[idx_v.at[buf, b]], rows_v.at[buf, b], gsem
            )
            for b in range(GRP)
        ]

    pending = stage(0, 0)
    wb = None
    for g in range(NGRP):
        if wb is not None:
            wb.wait()
        nxt = None
        if g + 1 < NGRP:
            nxt = stage(g + 1, (g + 1) % 2)
        for cp in pending:
            cp.wait()
        base = wid * B_PER_W + g * GRP
        wb = pltpu.async_copy(rows_v.at[g % 2], out_hbm.at[pl.ds(base, GRP)], wsem)
        pending = nxt
    wb.wait()


@functools.cache
def _sc_gather():
    return pl.kernel(
        _sc_gather_body,
        out_type=jax.ShapeDtypeStruct((B, V, 2 * H), jnp.float32),
        mesh=plsc.VectorSubcoreMesh(core_axis_name="c", subcore_axis_name="s"),
        scratch_types=[
            pltpu.VMEM((2, GRP, V), jnp.int32),
            pltpu.VMEM((2, GRP, V, 2 * H), jnp.float32),
            pltpu.SemaphoreType.DMA,
            pltpu.SemaphoreType.DMA,
        ],
    )


BB = 128


def _mm_body(x_ref, mask_ref, e2_ref, bias_ref, out_ref):
    b = bias_ref[...]
    par = (x_ref[...] & 1).astype(jnp.float32)
    dn = (((1,), (0,)), ((), ()))
    for k in range(BB):
        pk = par[k:k + 1, :]
        mk = mask_ref[k]
        m1 = mk * pk
        m0 = mk - m1
        acc = lax.dot_general(
            m0.astype(jnp.bfloat16),
            e2_ref[k, :, :H].astype(jnp.bfloat16),
            dimension_numbers=dn,
            preferred_element_type=jnp.float32,
        ) + lax.dot_general(
            m1.astype(jnp.bfloat16),
            e2_ref[k, :, H:].astype(jnp.bfloat16),
            dimension_numbers=dn,
            preferred_element_type=jnp.float32,
        )
        out_ref[k] = acc + b


def _mm(x, mask, e2, bias, interpret=False):
    return pl.pallas_call(
        _mm_body,
        grid=(B // BB,),
        in_specs=[
            pl.BlockSpec((BB, V), lambda i: (i, 0)),
            pl.BlockSpec((BB, V, V), lambda i: (i, 0, 0)),
            pl.BlockSpec((BB, V, 2 * H), lambda i: (i, 0, 0)),
            pl.BlockSpec((V, H), lambda i: (0, 0)),
        ],
        out_specs=pl.BlockSpec((BB, V, H), lambda i: (i, 0, 0)),
        out_shape=jax.ShapeDtypeStruct((B, V, H), jnp.float32),
        compiler_params=pltpu.CompilerParams(
            dimension_semantics=("parallel",),
        ),
        interpret=interpret,
    )(x, mask, e2, bias)


def kernel(x, mask, emb_table, bias):
    xi = x.astype(jnp.int32)
    pos = jnp.arange(V, dtype=jnp.int32) * C
    idx_phys = (xi + pos[None, :]) >> 1
    table2 = emb_table.reshape(C * V // 2, 2 * H)
    e2 = _sc_gather()(table2, idx_phys)
    out = _mm(xi, mask, e2, bias)
    return lax.optimization_barrier(out)
